# baseline (speedup 1.0000x reference)
import functools

import jax
import jax.numpy as jnp
from jax import lax
from jax.experimental import pallas as pl
from jax.experimental.pallas import tpu as pltpu
from jax.experimental.pallas import tpu_sc as plsc

L_EDGES = 16384
LANES = 16
IBLK = 128


def _make_gather_kernel():
    info = plsc.get_sparse_core_info()
    nc, ns = info.num_cores, info.num_subcores
    nw = nc * ns
    chunk = L_EDGES // nw
    q_per_w = chunk // IBLK

    @functools.partial(
        pl.kernel,
        mesh=plsc.VectorSubcoreMesh(core_axis_name="c", subcore_axis_name="s"),
        compiler_params=pltpu.CompilerParams(use_tc_tiling_on_sc=False),
        out_type=jax.ShapeDtypeStruct((12, nw, q_per_w, IBLK, LANES),
                                      jnp.float32),
        scratch_types=[
            pltpu.VMEM((12, q_per_w, IBLK), jnp.int32),
            pltpu.VMEM((12, q_per_w, IBLK, LANES), jnp.float32),
            pltpu.SemaphoreType.DMA,
        ],
    )
    def gather_kernel(rows_hbm, rs_hbm, ss_hbm, ps_hbm, phi_hbm,
                      out_hbm, ridx_v, g_v, sem):
        wid = lax.axis_index("s") * nc + lax.axis_index("c")
        qbase = wid * q_per_w

        loads = [
            pltpu.async_copy(
                rows_hbm.at[r, pl.ds(qbase, q_per_w), :], ridx_v.at[r], sem)
            for r in range(12)
        ]
        for cp in loads:
            cp.wait()

        tbls = (rs_hbm, rs_hbm, ss_hbm, ss_hbm, ps_hbm, ps_hbm,
                phi_hbm, phi_hbm, phi_hbm, phi_hbm, phi_hbm, phi_hbm)
        gathers = []
        for r in range(12):
            for q in range(q_per_w):
                gathers.append(pltpu.async_copy(
                    tbls[r].at[ridx_v.at[r, q]], g_v.at[r, q], sem))
        for cp in gathers:
            cp.wait()

        stores = [
            pltpu.async_copy(g_v.at[r], out_hbm.at[r, wid], sem)
            for r in range(12)
        ]
        for cp in stores:
            cp.wait()

    return gather_kernel


def _elbo_body(params_ref, w_ref, lanes_ref, g_ref, out_ref):
    eps = 1e-12
    R = params_ref[0]
    inv2T = params_ref[1]
    alpha = params_ref[2]
    log_norm = params_ref[3]
    klt = params_ref[4]

    lanes = lanes_ref[...]
    rows = g_ref[...]
    onehot = lanes[:, None, :] == lax.broadcasted_iota(
        jnp.int32, rows.shape, 1)
    g = jnp.sum(jnp.where(onehot, rows, 0.0), axis=1)

    r1_raw = g[0]
    r2_raw = g[1]
    s1_raw = g[2]
    s2_raw = g[3]
    p1_raw = g[4]
    p2_raw = g[5]
    m1x, m1y, m1z = g[6], g[7], g[8]
    m2x, m2y, m2z = g[9], g[10], g[11]

    r_i = R / (1.0 + jnp.exp(-r1_raw))
    r_j = R / (1.0 + jnp.exp(-r2_raw))

    n1 = jnp.sqrt(m1x * m1x + m1y * m1y + m1z * m1z) + eps
    n2 = jnp.sqrt(m2x * m2x + m2y * m2y + m2z * m2z) + eps
    dot = m1x * m2x + m1y * m2y + m1z * m2z
    cos_dphi = jnp.clip(dot / (n1 * n2), -1.0, 1.0)

    e_i = jnp.exp(r_i)
    e_j = jnp.exp(r_j)
    ei_inv = 1.0 / e_i
    ej_inv = 1.0 / e_j
    cosh_i = 0.5 * (e_i + ei_inv)
    sinh_i = 0.5 * (e_i - ei_inv)
    cosh_j = 0.5 * (e_j + ej_inv)
    sinh_j = 0.5 * (e_j - ej_inv)

    ch = cosh_i * cosh_j - sinh_i * sinh_j * cos_dphi
    ch = jnp.maximum(ch, 1.0 + 1e-7)
    d = jnp.log(ch + jnp.sqrt(ch * ch - 1.0))

    u = (d - R) * inv2T
    p = 1.0 / (1.0 + jnp.exp(u))
    p = jnp.clip(p, eps, 1.0 - eps)
    edges = jnp.where(w_ref[...] > 0, 1.0, 0.0)

    ea_i = jnp.exp(alpha * r_i)
    ea_j = jnp.exp(alpha * r_j)
    sinh_ai = 0.5 * (ea_i - 1.0 / ea_i)
    sinh_aj = 0.5 * (ea_j - 1.0 / ea_j)
    log_r_i = jnp.log(alpha * sinh_ai + eps) - log_norm
    log_r_j = jnp.log(alpha * sinh_aj + eps) - log_norm

    s_i = jnp.exp(s1_raw) + jnp.exp(p1_raw)
    s_j = jnp.exp(s2_raw) + jnp.exp(p2_raw)

    ll = (edges * jnp.log(p) + (1.0 - edges) * jnp.log1p(-p)
          + log_r_i + log_r_j - 1e-3 * (s_i + s_j) - klt)
    out_ref[...] = ll


def kernel(idx1, idx2, weights, rs_loc, rs_scale, phis_loc, phis_scale,
           R_loc, R_scale, T, alpha_loc, alpha_scale):
    eps = 1e-12

    i1 = idx1.astype(jnp.int32)
    i2 = idx2.astype(jnp.int32)
    gidx = jnp.stack([i1, i2, i1, i2, i1, i2,
                      i1 * 3, i1 * 3 + 1, i1 * 3 + 2,
                      i2 * 3, i2 * 3 + 1, i2 * 3 + 2])
    rows = lax.shift_right_logical(gidx, 4).reshape(12, -1, IBLK)
    lanes = jnp.bitwise_and(gidx, 15)

    gathered = _make_gather_kernel()(
        rows,
        rs_loc.reshape(-1, LANES), rs_scale.reshape(-1, LANES),
        phis_scale.reshape(-1, LANES), phis_loc.reshape(-1, LANES))
    gathered = gathered.reshape(12, L_EDGES, LANES).transpose(0, 2, 1)

    R = jnp.exp(R_loc)
    T_x = jnp.exp(T)
    T_s = T_x[0] / (T_x[0] + T_x[1])
    alpha = jnp.exp(alpha_loc)
    log_norm = jnp.log(jnp.cosh(alpha * R) - 1.0 + eps)
    kl_glob = (0.5 * (R_loc ** 2 + jnp.exp(R_scale) ** 2)
               + 0.5 * (alpha_loc ** 2 + jnp.exp(alpha_scale) ** 2))
    params = jnp.stack([R, 1.0 / (2.0 * T_s + eps), alpha, log_norm,
                        kl_glob / L_EDGES, 0.0, 0.0, 0.0])

    return pl.pallas_call(
        _elbo_body,
        out_shape=jax.ShapeDtypeStruct((L_EDGES,), jnp.float32),
        in_specs=[
            pl.BlockSpec(memory_space=pltpu.SMEM),
            pl.BlockSpec(memory_space=pltpu.VMEM),
            pl.BlockSpec(memory_space=pltpu.VMEM),
            pl.BlockSpec(memory_space=pltpu.VMEM),
        ],
    )(params, weights, lanes, gathered)


# speedup vs baseline: 1.0007x; 1.0007x over previous
import functools

import jax
import jax.numpy as jnp
from jax import lax
from jax.experimental import pallas as pl
from jax.experimental.pallas import tpu as pltpu
from jax.experimental.pallas import tpu_sc as plsc

L_EDGES = 16384
LANES = 16
IBLK = 128


def _make_gather_kernel():
    info = plsc.get_sparse_core_info()
    nc, ns = info.num_cores, info.num_subcores
    nw = nc * ns
    chunk = L_EDGES // nw
    q_per_w = chunk // IBLK

    @functools.partial(
        pl.kernel,
        mesh=plsc.VectorSubcoreMesh(core_axis_name="c", subcore_axis_name="s"),
        compiler_params=pltpu.CompilerParams(use_tc_tiling_on_sc=False),
        out_type=jax.ShapeDtypeStruct((12, nw, chunk, LANES), jnp.float32),
        scratch_types=[
            pltpu.VMEM((12, chunk), jnp.int32),
            pltpu.VMEM((12, chunk, LANES), jnp.float32),
            pltpu.SemaphoreType.DMA,
        ],
    )
    def gather_kernel(rows_hbm, rs_hbm, ss_hbm, ps_hbm, phi_hbm,
                      out_hbm, ridx_v, g_v, sem):
        wid = lax.axis_index("s") * nc + lax.axis_index("c")

        loads = [
            pltpu.async_copy(
                rows_hbm.at[r, pl.ds(wid * chunk, chunk)], ridx_v.at[r], sem)
            for r in range(12)
        ]
        for cp in loads:
            cp.wait()

        tbls = (rs_hbm, rs_hbm, ss_hbm, ss_hbm, ps_hbm, ps_hbm,
                phi_hbm, phi_hbm, phi_hbm, phi_hbm, phi_hbm, phi_hbm)
        gathers = [
            pltpu.async_copy(tbls[r].at[ridx_v.at[r]], g_v.at[r], sem)
            for r in range(12)
        ]
        for cp in gathers:
            cp.wait()

        stores = [
            pltpu.async_copy(g_v.at[r], out_hbm.at[r, wid], sem)
            for r in range(12)
        ]
        for cp in stores:
            cp.wait()

    return gather_kernel


def _elbo_body(params_ref, w_ref, lanes_ref, g_ref, out_ref):
    eps = 1e-12
    R = params_ref[0]
    inv2T = params_ref[1]
    alpha = params_ref[2]
    log_norm = params_ref[3]
    klt = params_ref[4]

    lanes = lanes_ref[...]
    rows = g_ref[...]
    onehot = lanes[:, None, :] == lax.broadcasted_iota(
        jnp.int32, rows.shape, 1)
    g = jnp.sum(jnp.where(onehot, rows, 0.0), axis=1)

    r1_raw = g[0]
    r2_raw = g[1]
    s1_raw = g[2]
    s2_raw = g[3]
    p1_raw = g[4]
    p2_raw = g[5]
    m1x, m1y, m1z = g[6], g[7], g[8]
    m2x, m2y, m2z = g[9], g[10], g[11]

    r_i = R / (1.0 + jnp.exp(-r1_raw))
    r_j = R / (1.0 + jnp.exp(-r2_raw))

    n1 = jnp.sqrt(m1x * m1x + m1y * m1y + m1z * m1z) + eps
    n2 = jnp.sqrt(m2x * m2x + m2y * m2y + m2z * m2z) + eps
    dot = m1x * m2x + m1y * m2y + m1z * m2z
    cos_dphi = jnp.clip(dot / (n1 * n2), -1.0, 1.0)

    e_i = jnp.exp(r_i)
    e_j = jnp.exp(r_j)
    ei_inv = 1.0 / e_i
    ej_inv = 1.0 / e_j
    cosh_i = 0.5 * (e_i + ei_inv)
    sinh_i = 0.5 * (e_i - ei_inv)
    cosh_j = 0.5 * (e_j + ej_inv)
    sinh_j = 0.5 * (e_j - ej_inv)

    ch = cosh_i * cosh_j - sinh_i * sinh_j * cos_dphi
    ch = jnp.maximum(ch, 1.0 + 1e-7)
    d = jnp.log(ch + jnp.sqrt(ch * ch - 1.0))

    u = (d - R) * inv2T
    p = 1.0 / (1.0 + jnp.exp(u))
    p = jnp.clip(p, eps, 1.0 - eps)
    edges = jnp.where(w_ref[...] > 0, 1.0, 0.0)

    ea_i = jnp.exp(alpha * r_i)
    ea_j = jnp.exp(alpha * r_j)
    sinh_ai = 0.5 * (ea_i - 1.0 / ea_i)
    sinh_aj = 0.5 * (ea_j - 1.0 / ea_j)
    log_r_i = jnp.log(alpha * sinh_ai + eps) - log_norm
    log_r_j = jnp.log(alpha * sinh_aj + eps) - log_norm

    s_i = jnp.exp(s1_raw) + jnp.exp(p1_raw)
    s_j = jnp.exp(s2_raw) + jnp.exp(p2_raw)

    ll = (edges * jnp.log(p) + (1.0 - edges) * jnp.log1p(-p)
          + log_r_i + log_r_j - 1e-3 * (s_i + s_j) - klt)
    out_ref[...] = ll


def kernel(idx1, idx2, weights, rs_loc, rs_scale, phis_loc, phis_scale,
           R_loc, R_scale, T, alpha_loc, alpha_scale):
    eps = 1e-12

    i1 = idx1.astype(jnp.int32)
    i2 = idx2.astype(jnp.int32)
    gidx = jnp.stack([i1, i2, i1, i2, i1, i2,
                      i1 * 3, i1 * 3 + 1, i1 * 3 + 2,
                      i2 * 3, i2 * 3 + 1, i2 * 3 + 2])
    rows = lax.shift_right_logical(gidx, 4)
    lanes = jnp.bitwise_and(gidx, 15)

    gathered = _make_gather_kernel()(
        rows,
        rs_loc.reshape(-1, LANES), rs_scale.reshape(-1, LANES),
        phis_scale.reshape(-1, LANES), phis_loc.reshape(-1, LANES))
    gathered = gathered.reshape(12, L_EDGES, LANES).transpose(0, 2, 1)

    R = jnp.exp(R_loc)
    T_x = jnp.exp(T)
    T_s = T_x[0] / (T_x[0] + T_x[1])
    alpha = jnp.exp(alpha_loc)
    log_norm = jnp.log(jnp.cosh(alpha * R) - 1.0 + eps)
    kl_glob = (0.5 * (R_loc ** 2 + jnp.exp(R_scale) ** 2)
               + 0.5 * (alpha_loc ** 2 + jnp.exp(alpha_scale) ** 2))
    params = jnp.stack([R, 1.0 / (2.0 * T_s + eps), alpha, log_norm,
                        kl_glob / L_EDGES, 0.0, 0.0, 0.0])

    return pl.pallas_call(
        _elbo_body,
        out_shape=jax.ShapeDtypeStruct((L_EDGES,), jnp.float32),
        in_specs=[
            pl.BlockSpec(memory_space=pltpu.SMEM),
            pl.BlockSpec(memory_space=pltpu.VMEM),
            pl.BlockSpec(memory_space=pltpu.VMEM),
            pl.BlockSpec(memory_space=pltpu.VMEM),
        ],
    )(params, weights, lanes, gathered)
